# baseline (speedup 1.0000x reference)
import functools

import jax
import jax.numpy as jnp
from jax.experimental import pallas as pl
from jax.experimental.pallas import tpu as pltpu

N_NODES = 50000
N_REL = 3
N_GRAPHS = 512
CE = 1000
CN = 2000


def _embed_kernel(x_ref, se_ref, ce_ref, pw_ref, pb_ref, out_ref):
    x0 = x_ref[:, 0:1]
    x1 = x_ref[:, 1:2]
    iota_s = jax.lax.broadcasted_iota(jnp.int32, (CN, 16), 1)
    oh_s = (x0 == iota_s).astype(jnp.float32)
    oh_c = (x1 == iota_s).astype(jnp.float32)
    emb_s = jnp.dot(oh_s, se_ref[...], preferred_element_type=jnp.float32)
    emb_c = jnp.dot(oh_c, ce_ref[...], preferred_element_type=jnp.float32)
    h = (jnp.dot(emb_s, pw_ref[0:8, :], preferred_element_type=jnp.float32)
         + jnp.dot(emb_c, pw_ref[8:16, :], preferred_element_type=jnp.float32)
         + pb_ref[...])
    h = jnp.maximum(h, 0.0)
    ones = jnp.ones((CN, 1), jnp.float32)
    zeros = jnp.zeros((CN, 31), jnp.float32)
    out_ref[...] = jnp.concatenate([h, ones, zeros], axis=1)


DST_CHUNKS = ((0, 20000), (20000, 20000), (40000, 10000))


def _edge_kernel(eb_ref, tab_ref, acc_ref, *, base, nd):
    @pl.when(pl.program_id(0) == 0)
    def _init():
        acc_ref[...] = jnp.zeros_like(acc_ref)

    def body(i, carry):
        s = eb_ref[i, 0]
        d = eb_ref[i, 1]
        t = eb_ref[i, 2]
        off = d - base
        valid = jnp.logical_and(off >= 0, off < nd)

        @pl.when(valid)
        def _acc():
            row = tab_ref[pl.ds(s, 1), :]
            j = t * nd + off
            acc_ref[pl.ds(j, 1), :] = acc_ref[pl.ds(j, 1), :] + row

        return carry

    jax.lax.fori_loop(0, CE, body, 0)


def _combine_kernel(h_ref, s_ref, cnt_ref, wrel_ref, wroot_ref, b_ref,
                    out_ref, *, width):
    h = h_ref[:, 0:width]
    out = jnp.dot(h, wroot_ref[...], preferred_element_type=jnp.float32)
    out = out + b_ref[...]
    for r in range(N_REL):
        sr = s_ref[r, :, 0:width]
        inv = 1.0 / jnp.maximum(cnt_ref[:, r:r + 1], 1.0)
        out = out + jnp.dot(sr * inv, wrel_ref[r],
                            preferred_element_type=jnp.float32)
    out_ref[...] = jnp.maximum(out, 0.0)


def _pool_kernel(b_ref, h_ref, psum_ref, gcnt_ref):
    @pl.when(pl.program_id(0) == 0)
    def _init():
        psum_ref[...] = jnp.zeros_like(psum_ref)
        gcnt_ref[...] = jnp.zeros_like(gcnt_ref)

    iota_g = jax.lax.broadcasted_iota(jnp.int32, (CN, N_GRAPHS), 1)
    oh = (b_ref[...] == iota_g).astype(jnp.float32)
    dn = (((0,), (0,)), ((), ()))
    psum_ref[...] += jax.lax.dot_general(
        oh, h_ref[...], dimension_numbers=dn,
        preferred_element_type=jnp.float32)
    gcnt_ref[...] += jax.lax.dot_general(
        oh, jnp.ones((CN, 1), jnp.float32), dimension_numbers=dn,
        preferred_element_type=jnp.float32)


def _classify_kernel(psum_ref, gcnt_ref, cw_ref, cb_ref, out_ref):
    inv = 1.0 / jnp.maximum(gcnt_ref[...], 1.0)
    pooled = psum_ref[...] * inv
    out_ref[...] = (jnp.dot(pooled, cw_ref[...],
                            preferred_element_type=jnp.float32)
                    + cb_ref[...])


def _run_edge_pass(edges, table):
    grid = edges.shape[0] // CE
    parts = []
    for base, nd in DST_CHUNKS:
        acc = pl.pallas_call(
            functools.partial(_edge_kernel, base=base, nd=nd),
            grid=(grid,),
            in_specs=[
                pl.BlockSpec((CE, 3), lambda i: (i, 0),
                             memory_space=pltpu.SMEM),
                pl.BlockSpec((N_NODES, 64), lambda i: (0, 0)),
            ],
            out_specs=pl.BlockSpec((N_REL * nd, 64), lambda i: (0, 0)),
            out_shape=jax.ShapeDtypeStruct((N_REL * nd, 64), jnp.float32),
            compiler_params=pltpu.CompilerParams(
                dimension_semantics=("arbitrary",)),
        )(edges, table)
        parts.append(acc.reshape(N_REL, nd, 64))
    return jnp.concatenate(parts, axis=1)


def _run_combine(h, s3, cnt, wrel, wroot, b, width):
    grid = N_NODES // CN
    return pl.pallas_call(
        functools.partial(_combine_kernel, width=width),
        grid=(grid,),
        in_specs=[
            pl.BlockSpec((CN, 64), lambda i: (i, 0)),
            pl.BlockSpec((N_REL, CN, 64), lambda i: (0, i, 0)),
            pl.BlockSpec((CN, N_REL), lambda i: (i, 0)),
            pl.BlockSpec(wrel.shape, lambda i: (0, 0, 0)),
            pl.BlockSpec(wroot.shape, lambda i: (0, 0)),
            pl.BlockSpec((1, 64), lambda i: (0, 0)),
        ],
        out_specs=pl.BlockSpec((CN, 64), lambda i: (i, 0)),
        out_shape=jax.ShapeDtypeStruct((N_NODES, 64), jnp.float32),
        compiler_params=pltpu.CompilerParams(
            dimension_semantics=("arbitrary",)),
    )(h, s3, cnt, wrel, wroot, b)


def kernel(x, edge_index, edge_type, batch, se, ce, pre_w, pre_b,
           w_rel1, w_root1, b1, w_rel2, w_root2, b2, cls_w, cls_b):
    x = x.astype(jnp.int32)
    edges = jnp.concatenate(
        [edge_index.astype(jnp.int32), edge_type.astype(jnp.int32)[None, :]],
        axis=0).T

    h0_ext = pl.pallas_call(
        _embed_kernel,
        grid=(N_NODES // CN,),
        in_specs=[
            pl.BlockSpec((CN, 2), lambda i: (i, 0)),
            pl.BlockSpec((16, 8), lambda i: (0, 0)),
            pl.BlockSpec((16, 8), lambda i: (0, 0)),
            pl.BlockSpec((16, 32), lambda i: (0, 0)),
            pl.BlockSpec((1, 32), lambda i: (0, 0)),
        ],
        out_specs=pl.BlockSpec((CN, 64), lambda i: (i, 0)),
        out_shape=jax.ShapeDtypeStruct((N_NODES, 64), jnp.float32),
    )(x, se, ce, pre_w, pre_b.reshape(1, 32))

    s1 = _run_edge_pass(edges, h0_ext)
    cnt = s1[:, :, 32].T

    h1 = _run_combine(h0_ext, s1, cnt, w_rel1, w_root1,
                      b1.reshape(1, 64), 32)

    s2 = _run_edge_pass(edges, h1)
    h2 = _run_combine(h1, s2, cnt, w_rel2, w_root2, b2.reshape(1, 64), 64)

    psum, gcnt = pl.pallas_call(
        _pool_kernel,
        grid=(N_NODES // CN,),
        in_specs=[
            pl.BlockSpec((CN, 1), lambda i: (i, 0)),
            pl.BlockSpec((CN, 64), lambda i: (i, 0)),
        ],
        out_specs=[
            pl.BlockSpec((N_GRAPHS, 64), lambda i: (0, 0)),
            pl.BlockSpec((N_GRAPHS, 1), lambda i: (0, 0)),
        ],
        out_shape=[
            jax.ShapeDtypeStruct((N_GRAPHS, 64), jnp.float32),
            jax.ShapeDtypeStruct((N_GRAPHS, 1), jnp.float32),
        ],
        compiler_params=pltpu.CompilerParams(
            dimension_semantics=("arbitrary",)),
    )(batch.astype(jnp.int32).reshape(N_NODES, 1), h2)

    logits = pl.pallas_call(
        _classify_kernel,
        in_specs=[
            pl.BlockSpec((N_GRAPHS, 64), lambda: (0, 0)),
            pl.BlockSpec((N_GRAPHS, 1), lambda: (0, 0)),
            pl.BlockSpec((64, 10), lambda: (0, 0)),
            pl.BlockSpec((1, 10), lambda: (0, 0)),
        ],
        out_specs=pl.BlockSpec((N_GRAPHS, 10), lambda: (0, 0)),
        out_shape=jax.ShapeDtypeStruct((N_GRAPHS, 10), jnp.float32),
    )(psum, gcnt, cls_w, cls_b.reshape(1, 10))
    return logits
